# Initial kernel scaffold; baseline (speedup 1.0000x reference)
import functools

import jax
import jax.numpy as jnp
from jax import lax
from jax.experimental import pallas as pl
from jax.experimental.pallas import tpu as pltpu
from jax.experimental.pallas import tpu_sc as plsc

NUM_EMB = 100000
D = 128
HALF = D // 2
B = 16384
NC = 2
NS = 16
NW = NC * NS
BPW = B // NW
CHUNK = 128
NCHUNK = BPW // CHUNK

_mesh = plsc.VectorSubcoreMesh(core_axis_name="c", subcore_axis_name="s")


@functools.partial(
    pl.kernel,
    mesh=_mesh,
    out_type=jax.ShapeDtypeStruct((2, B, HALF), jnp.float32),
    scratch_types=[
        pltpu.VMEM((NCHUNK, CHUNK), jnp.int32),
        pltpu.VMEM((CHUNK, D), jnp.float32),
        pltpu.VMEM((CHUNK, HALF), jnp.float32),
        pltpu.VMEM((CHUNK, HALF), jnp.float32),
        pltpu.SemaphoreType.DMA,
        pltpu.SemaphoreType.DMA,
    ],
)
def _gather_split(ids_hbm, table_hbm, out_hbm, idx_v, rows_v, re_v, im_v,
                  gsem, osem):
    wid = lax.axis_index("s") * NC + lax.axis_index("c")
    base = wid * BPW
    pltpu.sync_copy(ids_hbm.at[pl.ds(base, BPW)], idx_v.at[...])
    evens = lax.iota(jnp.int32, 16) * 2

    for c in range(NCHUNK):
        pltpu.async_copy(table_hbm.at[idx_v.at[c]], rows_v, gsem).wait()

        def row_body(r, _):
            for j in range(HALF // 16):
                cols = evens + (32 * j)
                row = jnp.full((16,), r, jnp.int32)
                re_v[r, pl.ds(16 * j, 16)] = plsc.load_gather(
                    rows_v, [row, cols])
                im_v[r, pl.ds(16 * j, 16)] = plsc.load_gather(
                    rows_v, [row, cols + 1])
            return 0

        lax.fori_loop(0, CHUNK, row_body, 0)

        off = base + c * CHUNK
        cp_r = pltpu.async_copy(re_v, out_hbm.at[0, pl.ds(off, CHUNK)], osem)
        cp_i = pltpu.async_copy(im_v, out_hbm.at[1, pl.ds(off, CHUNK)], osem)
        cp_r.wait()
        cp_i.wait()


def kernel(input_ids, table):
    out = _gather_split(input_ids.astype(jnp.int32), table)
    return out[0], out[1]


# speedup vs baseline: 4.7036x; 4.7036x over previous
import functools

import jax
import jax.numpy as jnp
from jax import lax
from jax.experimental import pallas as pl
from jax.experimental.pallas import tpu as pltpu
from jax.experimental.pallas import tpu_sc as plsc

NUM_EMB = 100000
D = 128
HALF = D // 2
B = 16384
NC = 2
NS = 16
NW = NC * NS
BPW = B // NW
CHUNK = 128
NCHUNK = BPW // CHUNK

_mesh = plsc.VectorSubcoreMesh(core_axis_name="c", subcore_axis_name="s")


@functools.partial(
    pl.kernel,
    mesh=_mesh,
    out_type=jax.ShapeDtypeStruct((2, B, HALF), jnp.float32),
    scratch_types=[
        pltpu.VMEM((BPW,), jnp.int32),
        pltpu.VMEM((CHUNK, D), jnp.float32),
        pltpu.VMEM((CHUNK, HALF), jnp.float32),
        pltpu.VMEM((CHUNK, HALF), jnp.float32),
        pltpu.SemaphoreType.DMA,
        pltpu.SemaphoreType.DMA,
    ],
    compiler_params=pltpu.CompilerParams(needs_layout_passes=False),
)
def _gather_split(ids_hbm, table_hbm, out_hbm, idx_v, rows_v, re_v, im_v,
                  gsem, osem):
    wid = lax.axis_index("s") * NC + lax.axis_index("c")
    base = wid * BPW
    pltpu.sync_copy(ids_hbm.at[pl.ds(base, BPW)], idx_v)
    evens = lax.iota(jnp.int32, 16) * 2

    for c in range(NCHUNK):
        pltpu.async_copy(
            table_hbm.at[idx_v.at[pl.ds(c * CHUNK, CHUNK)]],
            rows_v, gsem).wait()

        def row_body(r, _):
            row = jnp.full((16,), r, jnp.int32)
            for j in range(HALF // 16):
                cols = evens + (32 * j)
                re_v[r, pl.ds(16 * j, 16)] = plsc.load_gather(
                    rows_v, [row, cols])
                im_v[r, pl.ds(16 * j, 16)] = plsc.load_gather(
                    rows_v, [row, cols + 1])
            return 0

        lax.fori_loop(0, CHUNK, row_body, 0)

        off = base + c * CHUNK
        cp_r = pltpu.async_copy(re_v, out_hbm.at[0, pl.ds(off, CHUNK)], osem)
        cp_i = pltpu.async_copy(im_v, out_hbm.at[1, pl.ds(off, CHUNK)], osem)
        cp_r.wait()
        cp_i.wait()


def kernel(input_ids, table):
    out = _gather_split(input_ids.astype(jnp.int32), table)
    return out[0], out[1]


# speedup vs baseline: 6.2459x; 1.3279x over previous
import functools

import jax
import jax.numpy as jnp
from jax import lax
from jax.experimental import pallas as pl
from jax.experimental.pallas import tpu as pltpu
from jax.experimental.pallas import tpu_sc as plsc

NUM_EMB = 100000
D = 128
HALF = D // 2
B = 16384
NC = 2
NS = 16
NW = NC * NS
BPW = B // NW
CHUNK = 128
NCHUNK = BPW // CHUNK

_mesh = plsc.VectorSubcoreMesh(core_axis_name="c", subcore_axis_name="s")


@functools.partial(
    pl.kernel,
    mesh=_mesh,
    out_type=jax.ShapeDtypeStruct((2, B, HALF), jnp.float32),
    scratch_types=(
        [pltpu.VMEM((BPW,), jnp.int32)]
        + [pltpu.VMEM((CHUNK, D), jnp.float32) for _ in range(2)]
        + [pltpu.VMEM((CHUNK, HALF), jnp.float32) for _ in range(4)]
        + [pltpu.SemaphoreType.DMA((NCHUNK,)), pltpu.SemaphoreType.DMA]
    ),
    compiler_params=pltpu.CompilerParams(needs_layout_passes=False),
)
def _gather_split(ids_hbm, table_hbm, out_hbm, idx_v,
                  rows0, rows1, re0, re1, im0, im1,
                  gsem, osem):
    rows = [rows0, rows1]
    res = [re0, re1]
    ims = [im0, im1]
    wid = lax.axis_index("s") * NC + lax.axis_index("c")
    base = wid * BPW
    pltpu.sync_copy(ids_hbm.at[pl.ds(base, BPW)], idx_v)

    def fire_gather(c):
        return pltpu.async_copy(
            table_hbm.at[idx_v.at[pl.ds(c * CHUNK, CHUNK)]],
            rows[c % 2], gsem.at[c])

    gathers = [None] * NCHUNK
    gathers[0] = fire_gather(0)
    gathers[1] = fire_gather(1)

    evens = lax.iota(jnp.int32, 16) * 2
    cols = [evens + 32 * j for j in range(HALF // 16)]

    writes = [None] * (2 * NCHUNK)
    for c in range(NCHUNK):
        gathers[c].wait()
        if c >= 2:
            writes[2 * (c - 2)].wait()
            writes[2 * (c - 2) + 1].wait()
        rv = rows[c % 2]
        re_v = res[c % 2]
        im_v = ims[c % 2]

        @plsc.parallel_loop(0, CHUNK, unroll=4)
        def body(r):
            row = jnp.full((16,), r, jnp.int32)
            for j in range(HALF // 16):
                re_v[r, pl.ds(16 * j, 16)] = plsc.load_gather(
                    rv, [row, cols[j]])
                im_v[r, pl.ds(16 * j, 16)] = plsc.load_gather(
                    rv, [row, cols[j] + 1])

        if c + 2 < NCHUNK:
            gathers[c + 2] = fire_gather(c + 2)
        off = base + c * CHUNK
        writes[2 * c] = pltpu.async_copy(
            re_v, out_hbm.at[0, pl.ds(off, CHUNK)], osem)
        writes[2 * c + 1] = pltpu.async_copy(
            im_v, out_hbm.at[1, pl.ds(off, CHUNK)], osem)

    for c in range(NCHUNK - 2, NCHUNK):
        writes[2 * c].wait()
        writes[2 * c + 1].wait()


def kernel(input_ids, table):
    out = _gather_split(input_ids.astype(jnp.int32), table)
    return out[0], out[1]


# speedup vs baseline: 7.8202x; 1.2520x over previous
import functools

import jax
import jax.numpy as jnp
from jax import lax
from jax.experimental import pallas as pl
from jax.experimental.pallas import tpu as pltpu
from jax.experimental.pallas import tpu_sc as plsc

NUM_EMB = 100000
D = 128
HALF = D // 2
B = 16384
NC = 2
NS = 16
NW = NC * NS
BPW = B // NW
CHUNK = 128
NCHUNK = BPW // CHUNK

_mesh = plsc.VectorSubcoreMesh(core_axis_name="c", subcore_axis_name="s")


@functools.partial(
    pl.kernel,
    mesh=_mesh,
    out_type=(jax.ShapeDtypeStruct((B, HALF), jnp.float32),
              jax.ShapeDtypeStruct((B, HALF), jnp.float32)),
    scratch_types=(
        [pltpu.VMEM((BPW,), jnp.int32)]
        + [pltpu.VMEM((CHUNK, D), jnp.float32) for _ in range(2)]
        + [pltpu.VMEM((CHUNK, HALF), jnp.float32) for _ in range(4)]
        + [pltpu.SemaphoreType.DMA((NCHUNK,)), pltpu.SemaphoreType.DMA]
    ),
    compiler_params=pltpu.CompilerParams(needs_layout_passes=False),
)
def _gather_split(ids_hbm, table_hbm, re_hbm, im_hbm, idx_v,
                  rows0, rows1, re0, re1, im0, im1,
                  gsem, osem):
    rows = [rows0, rows1]
    res = [re0, re1]
    ims = [im0, im1]
    wid = lax.axis_index("s") * NC + lax.axis_index("c")
    base = wid * BPW
    pltpu.sync_copy(ids_hbm.at[pl.ds(base, BPW)], idx_v)

    def fire_gather(c):
        return pltpu.async_copy(
            table_hbm.at[idx_v.at[pl.ds(c * CHUNK, CHUNK)]],
            rows[c % 2], gsem.at[c])

    gathers = [None] * NCHUNK
    gathers[0] = fire_gather(0)
    gathers[1] = fire_gather(1)

    evens = lax.iota(jnp.int32, 16) * 2
    cols = [evens + 32 * j for j in range(HALF // 16)]

    writes = [None] * (2 * NCHUNK)
    for c in range(NCHUNK):
        gathers[c].wait()
        if c >= 2:
            writes[2 * (c - 2)].wait()
            writes[2 * (c - 2) + 1].wait()
        rv = rows[c % 2]
        re_v = res[c % 2]
        im_v = ims[c % 2]

        @plsc.parallel_loop(0, CHUNK, unroll=4)
        def body(r):
            row = jnp.full((16,), r, jnp.int32)
            for j in range(HALF // 16):
                re_v[r, pl.ds(16 * j, 16)] = plsc.load_gather(
                    rv, [row, cols[j]])
                im_v[r, pl.ds(16 * j, 16)] = plsc.load_gather(
                    rv, [row, cols[j] + 1])

        if c + 2 < NCHUNK:
            gathers[c + 2] = fire_gather(c + 2)
        off = base + c * CHUNK
        writes[2 * c] = pltpu.async_copy(
            re_v, re_hbm.at[pl.ds(off, CHUNK)], osem)
        writes[2 * c + 1] = pltpu.async_copy(
            im_v, im_hbm.at[pl.ds(off, CHUNK)], osem)

    for c in range(NCHUNK - 2, NCHUNK):
        writes[2 * c].wait()
        writes[2 * c + 1].wait()


def kernel(input_ids, table):
    return _gather_split(input_ids.astype(jnp.int32), table)
